# baseline (speedup 1.0000x reference)
import functools

import jax
import jax.numpy as jnp
from jax import lax
from jax.experimental import pallas as pl
from jax.experimental.pallas import tpu as pltpu
from jax.experimental.pallas import tpu_sc as plsc

D = 768
B = 4
S = 4096
NC = 2
NS = 16
NW = NC * NS
S_PER_W = S // NW
C = 8
NCHUNK = S_PER_W // C
NBUF = 4
LANES = 16
VECS_PER_ROW = D // LANES


def _emb_kernel(x_hbm, tok_hbm, pos_hbm, out_hbm,
                pos_v, tok_v, idx_v, idxh_v,
                p0, p1, p2, p3, g0, g1, g2, g3, s0_, s1_, s2_, s3_):
    psems = (p0, p1, p2, p3)
    gsems = (g0, g1, g2, g3)
    ssems = (s0_, s1_, s2_, s3_)
    wid = lax.axis_index("s") * NC + lax.axis_index("c")
    s_base = wid * S_PER_W

    head_cps = [
        pltpu.async_copy(x_hbm.at[b, pl.ds(s_base, 2 * C)], idxh_v.at[b],
                         gsems[b])
        for b in range(B)
    ]
    full_cps = [
        pltpu.async_copy(x_hbm.at[b, pl.ds(s_base, S_PER_W)], idx_v.at[b],
                         ssems[b])
        for b in range(B)
    ]
    for cp in head_cps:
        cp.wait()

    def fire(cc, q, idx_ref=None, idx_off=None):
        s0 = s_base + cc * C
        idx_ref = idx_v if idx_ref is None else idx_ref
        idx_off = cc * C if idx_off is None else idx_off
        pltpu.async_copy(pos_hbm.at[pl.ds(s0, C)], pos_v.at[q], psems[q])
        for b in range(B):
            pltpu.async_copy(tok_hbm.at[idx_ref.at[b, pl.ds(idx_off, C)]],
                             tok_v.at[q, b], gsems[q])

    def wait_fire(q):
        pltpu.make_async_copy(pos_hbm.at[pl.ds(0, C)], pos_v.at[q],
                              psems[q]).wait()
        for b in range(B):
            pltpu.make_async_copy(tok_hbm.at[idx_v.at[b, pl.ds(0, C)]],
                                  tok_v.at[q, b], gsems[q]).wait()

    def fire_store(cc, q):
        s0 = s_base + cc * C
        for b in range(B):
            pltpu.async_copy(tok_v.at[q, b], out_hbm.at[b, pl.ds(s0, C)],
                             ssems[q])

    def wait_store(q):
        for b in range(B):
            pltpu.make_async_copy(tok_v.at[q, b],
                                  out_hbm.at[b, pl.ds(0, C)],
                                  ssems[q]).wait()

    def adds(q):
        def add_row(r, _):
            for j in range(VECS_PER_ROW):
                sl = pl.ds(j * LANES, LANES)
                pv = pos_v[q, r, sl]
                for b in range(B):
                    tok_v[q, b, r, sl] = tok_v[q, b, r, sl] + pv
            return 0

        lax.fori_loop(0, C, add_row, 0)

    fire(0, 0, idxh_v, 0)
    fire(1, 1, idxh_v, C)
    for cp in full_cps:
        cp.wait()

    def body(i, carry):
        for k in range(NBUF):
            c = NBUF * i + k
            q = (k + 2) % NBUF
            wait_fire(k)
            cc2 = c + 2

            @pl.when(cc2 < NCHUNK)
            def _():
                @pl.when(cc2 >= NBUF)
                def _():
                    wait_store(q)

                fire(cc2, q)

            adds(k)
            fire_store(c, k)
        return carry

    lax.fori_loop(0, NCHUNK // NBUF, body, 0)
    for q in range(NBUF):
        wait_store(q)


@jax.jit
def _emb(x, token_table, pos_table):
    mesh = plsc.VectorSubcoreMesh(core_axis_name="c", subcore_axis_name="s")
    kern = functools.partial(
        pl.kernel,
        mesh=mesh,
        out_type=jax.ShapeDtypeStruct((B, S, D), jnp.float32),
        scratch_types=[
            pltpu.VMEM((NBUF, C, D), jnp.float32),
            pltpu.VMEM((NBUF, B, C, D), jnp.float32),
            pltpu.VMEM((B, S_PER_W), jnp.int32),
            pltpu.VMEM((B, 2 * C), jnp.int32),
        ] + [pltpu.SemaphoreType.DMA] * 12,
    )(_emb_kernel)
    return kern(x, token_table, pos_table)


def kernel(x, token_table, pos_table):
    return _emb(x.astype(jnp.int32), token_table, pos_table)


# speedup vs baseline: 1.0024x; 1.0024x over previous
import functools

import jax
import jax.numpy as jnp
from jax import lax
from jax.experimental import pallas as pl
from jax.experimental.pallas import tpu as pltpu
from jax.experimental.pallas import tpu_sc as plsc

D = 768
B = 4
S = 4096
NC = 2
NS = 16
NW = NC * NS
S_PER_W = S // NW
C = 8
NCHUNK = S_PER_W // C
NBUF = 4
LANES = 16
VECS_PER_ROW = D // LANES


def _emb_kernel(x_hbm, tok_hbm, pos_hbm, out_hbm,
                pos_v, tok_v, idx_v,
                p0, p1, p2, p3, g0, g1, g2, g3, s0_, s1_, s2_, s3_):
    psems = (p0, p1, p2, p3)
    gsems = (g0, g1, g2, g3)
    ssems = (s0_, s1_, s2_, s3_)
    wid = lax.axis_index("s") * NC + lax.axis_index("c")
    s_base = wid * S_PER_W

    idx_cps = [
        pltpu.async_copy(x_hbm.at[b, pl.ds(s_base, S_PER_W)], idx_v.at[b],
                         gsems[b])
        for b in range(B)
    ]
    for cp in idx_cps:
        cp.wait()

    def fire(cc, q):
        s0 = s_base + cc * C
        pltpu.async_copy(pos_hbm.at[pl.ds(s0, C)], pos_v.at[q], psems[q])
        for b in range(B):
            pltpu.async_copy(tok_hbm.at[idx_v.at[b, pl.ds(cc * C, C)]],
                             tok_v.at[q, b], gsems[q])

    def wait_fire(q):
        pltpu.make_async_copy(pos_hbm.at[pl.ds(0, C)], pos_v.at[q],
                              psems[q]).wait()
        for b in range(B):
            pltpu.make_async_copy(tok_hbm.at[idx_v.at[b, pl.ds(0, C)]],
                                  tok_v.at[q, b], gsems[q]).wait()

    def fire_store(cc, q):
        s0 = s_base + cc * C
        for b in range(B):
            pltpu.async_copy(tok_v.at[q, b], out_hbm.at[b, pl.ds(s0, C)],
                             ssems[q])

    def wait_store(q):
        for b in range(B):
            pltpu.make_async_copy(tok_v.at[q, b],
                                  out_hbm.at[b, pl.ds(0, C)],
                                  ssems[q]).wait()

    def adds(q):
        def add_row(r, _):
            for j in range(VECS_PER_ROW):
                sl = pl.ds(j * LANES, LANES)
                pv = pos_v[q, r, sl]
                for b in range(B):
                    tok_v[q, b, r, sl] = tok_v[q, b, r, sl] + pv
            return 0

        lax.fori_loop(0, C, add_row, 0)

    fire(0, 0)
    fire(1, 1)

    def body(i, carry):
        for k in range(NBUF):
            c = NBUF * i + k
            q = (k + 2) % NBUF
            wait_fire(k)
            cc2 = c + 2

            @pl.when(cc2 < NCHUNK)
            def _():
                @pl.when(cc2 >= NBUF)
                def _():
                    wait_store(q)

                fire(cc2, q)

            adds(k)
            fire_store(c, k)
        return carry

    lax.fori_loop(0, NCHUNK // NBUF, body, 0)
    for q in range(NBUF):
        wait_store(q)


@jax.jit
def _emb(x, token_table, pos_table):
    mesh = plsc.VectorSubcoreMesh(core_axis_name="c", subcore_axis_name="s")
    kern = functools.partial(
        pl.kernel,
        mesh=mesh,
        out_type=jax.ShapeDtypeStruct((B, S, D), jnp.float32),
        scratch_types=[
            pltpu.VMEM((NBUF, C, D), jnp.float32),
            pltpu.VMEM((NBUF, B, C, D), jnp.float32),
            pltpu.VMEM((B, S_PER_W), jnp.int32),
        ] + [pltpu.SemaphoreType.DMA] * 12,
    )(_emb_kernel)
    return kern(x, token_table, pos_table)


def kernel(x, token_table, pos_table):
    return _emb(x.astype(jnp.int32), token_table, pos_table)
